# baseline (speedup 1.0000x reference)
import functools

import jax
import jax.numpy as jnp
from jax import lax
from jax.experimental import pallas as pl
from jax.experimental.pallas import tpu as pltpu
from jax.experimental.pallas import tpu_sc as plsc

GAMMA = 1.0

NC = 2
NS = 16
NW = NC * NS
LANES = 16

FRAC0 = 0.678


def _sc_body(ns0, ns1, g, chunk0, chunk1, col_stride, p_valid, d,
             idx_hbm, feat_hbm, out_hbm, idxl_v, idxr_v, idxfl_v, idxfr_v,
             rows_l, rows_r, rows_fl, rows_fr, out_stage, sem):
  cid = lax.axis_index("c")
  sid = lax.axis_index("s")
  wid = sid * NC + cid
  base = jnp.where(cid == 0, sid * chunk0, NS * chunk0 + sid * chunk1)
  nsteps = jnp.where(cid == 0, ns0, ns1)

  idx_bufs = (idxl_v, idxr_v, idxfl_v, idxfr_v)
  for c in range(4):
    pltpu.sync_copy(idx_hbm.at[pl.ds(c * col_stride + base, chunk0 + g)],
                    idx_bufs[c])

  row_bufs = (rows_l, rows_r, rows_fl, rows_fr)
  nq = d // LANES
  lane = lax.iota(jnp.int32, LANES)
  rots = [(lane + s) & (LANES - 1) for s in (8, 4, 2, 1)]
  dnums = lax.GatherDimensionNumbers(
      offset_dims=(), collapsed_slice_dims=(0,), start_index_map=(0,))

  def hsum(x):
    for perm in rots:
      rot = lax.gather(x, perm[:, None], dnums, slice_sizes=(1,),
                       mode=lax.GatherScatterMode.PROMISE_IN_BOUNDS)
      x = x + rot
    return x

  def gather_step(t, half):
    for c in range(4):
      pltpu.async_copy(feat_hbm.at[idx_bufs[c].at[pl.ds(t * g, g)]],
                       row_bufs[c].at[pl.ds(half, g)], sem)

  def wait_step(half):
    for c in range(4):
      pltpu.make_async_copy(feat_hbm.at[pl.ds(0, g)],
                            row_bufs[c].at[pl.ds(half, g)], sem).wait()

  unroll = 4

  def compute_step(t, half, acc):
    def pair_group(pg, acc2):
      for k in range(unroll):
        i = half + pg * unroll + k
        u = None
        v = None
        for q in range(nq):
          sl = pl.ds(q * LANES, LANES)
          lv = rows_l[i, sl]
          rv = rows_r[i, sl]
          flv = rows_fl[i, sl]
          frv = rows_fr[i, sl]
          a = jnp.abs(lv - rv)
          du = a - jnp.abs(lv - frv)
          dv = a - jnp.abs(flv - rv)
          u = du if u is None else u + du
          v = dv if v is None else v + dv
        su = hsum(u)
        sv = hsum(v)
        contrib = (jnp.maximum(GAMMA + su, 0.0) +
                   jnp.maximum(GAMMA + sv, 0.0))
        valid = (base + t * g + (pg * unroll + k)) < p_valid
        acc2 = acc2 + jnp.where(valid, contrib, 0.0)
      return acc2

    return lax.fori_loop(0, g // unroll, pair_group, acc)

  def body(t, acc):
    half = (t & 1) * g
    gather_step(t, half)
    wait_step(half)
    return compute_step(t, half, acc)

  acc = lax.fori_loop(0, nsteps, body,
                      jnp.zeros((LANES,), jnp.float32))

  out_stage[...] = jnp.where(lane == 0, acc, 0.0)
  pltpu.sync_copy(out_stage, out_hbm.at[wid])


def _round_up(x, m):
  return (x + m - 1) // m * m


def _build_sc_call(p_valid, v_rows, d, g):
  per_tile = (p_valid + NS - 1) // NS
  chunk0 = _round_up(int(per_tile * FRAC0), g)
  chunk1 = _round_up(per_tile - chunk0 + g, g)
  assert chunk0 >= chunk1
  total = NS * (chunk0 + chunk1)
  assert total >= p_valid
  col_stride = _round_up(NS * chunk0 + (NS - 1) * chunk1 + chunk0 + 2 * g, 8)
  mesh = plsc.VectorSubcoreMesh(core_axis_name="c", subcore_axis_name="s")
  body = functools.partial(_sc_body, chunk0 // g, chunk1 // g, g, chunk0,
                           chunk1, col_stride, p_valid, d)
  row_buf = pltpu.VMEM((2 * g, d), jnp.float32)
  idx_buf = pltpu.VMEM((chunk0 + g,), jnp.int32)
  return col_stride, pl.kernel(
      body,
      out_type=jax.ShapeDtypeStruct((NW, LANES), jnp.float32),
      mesh=mesh,
      scratch_types=[
          idx_buf, idx_buf, idx_buf, idx_buf,
          row_buf, row_buf, row_buf, row_buf,
          pltpu.VMEM((LANES,), jnp.float32),
          pltpu.SemaphoreType.DMA,
      ],
  )


def kernel(pairs, features):
  p, _ = pairs.shape
  v_rows, d = features.shape
  g = 64
  col_stride, call = _build_sc_call(p, v_rows, d, g)
  idx = jnp.zeros((4, col_stride), jnp.int32).at[:, :p].set(pairs.T)
  partials = call(idx.reshape(-1), features)
  return jnp.sum(partials) / v_rows


# speedup vs baseline: 1.4190x; 1.4190x over previous
import functools

import jax
import jax.numpy as jnp
from jax import lax
from jax.experimental import pallas as pl
from jax.experimental.pallas import tpu as pltpu
from jax.experimental.pallas import tpu_sc as plsc

GAMMA = 1.0

NC = 2
NS = 16
NW = NC * NS
LANES = 16

FRAC0 = 0.678


def _sc_body(ns0, ns1, g, chunk0, chunk1, p_valid, d,
             idx_hbm, feat_hbm, out_hbm, idx_v, rows, out_stage, sem):
  cid = lax.axis_index("c")
  sid = lax.axis_index("s")
  wid = sid * NC + cid
  base = jnp.where(cid == 0, sid * chunk0, NS * chunk0 + sid * chunk1)
  nsteps = jnp.where(cid == 0, ns0, ns1)
  g4 = 4 * g

  pltpu.sync_copy(idx_hbm.at[pl.ds(4 * base, 4 * chunk0 + g4)], idx_v)

  nq = d // LANES
  lane = lax.iota(jnp.int32, LANES)
  rots = [(lane + s) & (LANES - 1) for s in (8, 4, 2, 1)]
  dnums = lax.GatherDimensionNumbers(
      offset_dims=(), collapsed_slice_dims=(0,), start_index_map=(0,))

  def hsum(x):
    for perm in rots:
      rot = lax.gather(x, perm[:, None], dnums, slice_sizes=(1,),
                       mode=lax.GatherScatterMode.PROMISE_IN_BOUNDS)
      x = x + rot
    return x

  def gather_step(t, half4):
    pltpu.async_copy(feat_hbm.at[idx_v.at[pl.ds(t * g4, g4)]],
                     rows.at[pl.ds(half4, g4)], sem)

  def wait_step(half4):
    pltpu.make_async_copy(feat_hbm.at[pl.ds(0, g4)],
                          rows.at[pl.ds(half4, g4)], sem).wait()

  unroll = 4

  def compute_step(t, half4, acc):
    def pair_group(pg, acc2):
      for k in range(unroll):
        i = half4 + pg * unroll + k
        u = None
        v = None
        for q in range(nq):
          sl = pl.ds(q * LANES, LANES)
          lv = rows[i, sl]
          rv = rows[i + g, sl]
          flv = rows[i + 2 * g, sl]
          frv = rows[i + 3 * g, sl]
          a = jnp.abs(lv - rv)
          du = a - jnp.abs(lv - frv)
          dv = a - jnp.abs(flv - rv)
          u = du if u is None else u + du
          v = dv if v is None else v + dv
        su = hsum(u)
        sv = hsum(v)
        contrib = (jnp.maximum(GAMMA + su, 0.0) +
                   jnp.maximum(GAMMA + sv, 0.0))
        valid = (base + t * g + (pg * unroll + k)) < p_valid
        acc2 = acc2 + jnp.where(valid, contrib, 0.0)
      return acc2

    return lax.fori_loop(0, g // unroll, pair_group, acc)

  gather_step(0, 0)

  def body(t, acc):
    half4 = (t & 1) * g4
    wait_step(half4)
    gather_step(t + 1, g4 - half4)
    return compute_step(t, half4, acc)

  acc = lax.fori_loop(0, nsteps, body,
                      jnp.zeros((LANES,), jnp.float32))
  wait_step((nsteps & 1) * g4)

  out_stage[...] = jnp.where(lane == 0, acc, 0.0)
  pltpu.sync_copy(out_stage, out_hbm.at[wid])


def _round_up(x, m):
  return (x + m - 1) // m * m


def _build_sc_call(p_valid, v_rows, d, g):
  per_tile = (p_valid + NS - 1) // NS
  chunk0 = _round_up(int(per_tile * FRAC0), g)
  chunk1 = _round_up(per_tile - chunk0 + g, g)
  assert chunk0 >= chunk1
  total = NS * (chunk0 + chunk1)
  assert total >= p_valid
  mesh = plsc.VectorSubcoreMesh(core_axis_name="c", subcore_axis_name="s")
  body = functools.partial(_sc_body, chunk0 // g, chunk1 // g, g, chunk0,
                           chunk1, p_valid, d)
  return chunk0, chunk1, pl.kernel(
      body,
      out_type=jax.ShapeDtypeStruct((NW, LANES), jnp.float32),
      mesh=mesh,
      scratch_types=[
          pltpu.VMEM((4 * chunk0 + 4 * g,), jnp.int32),
          pltpu.VMEM((8 * g, d), jnp.float32),
          pltpu.VMEM((LANES,), jnp.float32),
          pltpu.SemaphoreType.DMA,
      ],
  )


def kernel(pairs, features):
  p, _ = pairs.shape
  v_rows, d = features.shape
  g = 64
  chunk0, chunk1, call = _build_sc_call(p, v_rows, d, g)
  total = NS * (chunk0 + chunk1)
  pp = jnp.zeros((total, 4), jnp.int32).at[:p].set(pairs)
  part0 = pp[:NS * chunk0].reshape(NS, chunk0 // g, g, 4)
  part1 = pp[NS * chunk0:].reshape(NS, chunk1 // g, g, 4)
  tail = 4 * (g + chunk0 - chunk1)
  idx = jnp.concatenate([
      part0.transpose(0, 1, 3, 2).reshape(-1),
      part1.transpose(0, 1, 3, 2).reshape(-1),
      jnp.zeros((tail,), jnp.int32),
  ])
  partials = call(idx, features)
  return jnp.sum(partials) / v_rows
